# baseline (speedup 1.0000x reference)
import jax
import jax.numpy as jnp
from jax import lax
from jax.experimental import pallas as pl
from jax.experimental.pallas import tpu as pltpu
from jax.experimental.pallas import tpu_sc as plsc

N_USER = 10000
N_ITEM = 40000
N = N_USER + N_ITEM
NP = 50048
D = 64
G = 16
E = 800000
B = 1024

NC = 2
NS = 16
L = 16

EPAD = 819200
ET = EPAD // NS
KB = 256
SB = 8
SKB = SB * KB
NSB = ET // SKB
NR_T = NP // NS

_mesh = plsc.VectorSubcoreMesh(
    core_axis_name="c", subcore_axis_name="s", num_cores=NC, num_subcores=NS)


def _spmm_body(ego_hbm, col_hbm, row_hbm, val_hbm, zz_hbm, out_hbm,
               colv, rowv, valv, rows0, rows1, tbl, acc,
               gsem0, gsem1, ssem0, ssem1):
    c = lax.axis_index("c")
    s = lax.axis_index("s")
    rows = (rows0, rows1)
    gsem = (gsem0, gsem1)
    ssem = (ssem0, ssem1)

    def scale(rbuf, vbase):
        def grp(g, carry):
            vv = valv[pl.ds(vbase + g * L, L)]
            for u in range(L):
                kk = g * L + u
                bv = lax.broadcast(vv[u], (L,))
                rbuf[kk, pl.ds(0, L)] = rbuf[kk, pl.ds(0, L)] * bv
            return carry
        lax.fori_loop(0, KB // L, grp, 0)

    for q in range(2):
        gbase = (2 * q + c) * NP
        pltpu.sync_copy(ego_hbm.at[pl.ds(gbase + s * NR_T, NR_T)],
                        tbl.at[pl.ds(s * NR_T, NR_T)])
        pltpu.sync_copy(zz_hbm, acc.at[pl.ds(s * NR_T, NR_T)])
        plsc.subcore_barrier()

        def superchunk(i, carry):
            ebase = s * ET + i * SKB
            pltpu.sync_copy(col_hbm.at[pl.ds(ebase, SKB)], colv)
            pltpu.sync_copy(row_hbm.at[pl.ds(ebase, SKB)], rowv)
            pltpu.sync_copy(val_hbm.at[pl.ds(ebase, SKB)], valv)

            pend_g = [None, None]
            pend_s = [None, None]

            def fire_gather(b):
                p = b & 1
                d = pltpu.make_async_copy(
                    tbl.at[colv.at[pl.ds(b * KB, KB)]], rows[p], gsem[p])
                d.start()
                pend_g[p] = d

            def drain_scale_scatter(b):
                p = b & 1
                pend_g[p].wait()
                scale(rows[p], b * KB)
                pend_s[p] = pltpu.async_copy(
                    rows[p], acc.at[rowv.at[pl.ds(b * KB, KB)]],
                    ssem[p], add=True)

            for b in range(SB):
                p = b & 1
                if b >= 2 and pend_s[p] is not None:
                    pend_s[p].wait()
                    pend_s[p] = None
                fire_gather(b)
                if b >= 1:
                    drain_scale_scatter(b - 1)
            drain_scale_scatter(SB - 1)
            for p in range(2):
                if pend_s[p] is not None:
                    pend_s[p].wait()
            return carry

        lax.fori_loop(0, NSB, superchunk, 0)
        plsc.subcore_barrier()

        pltpu.sync_copy(acc.at[pl.ds(s * NR_T, NR_T)],
                        out_hbm.at[pl.ds(gbase + s * NR_T, NR_T)])
        plsc.subcore_barrier()


@jax.jit
def _spmm(ego4, col4, row2, val, zz):
    return pl.kernel(
        _spmm_body,
        out_type=jax.ShapeDtypeStruct((4 * NP, G), jnp.float32),
        mesh=_mesh,
        scratch_types=[
            pltpu.VMEM((SKB,), jnp.int32),
            pltpu.VMEM((SKB,), jnp.int32),
            pltpu.VMEM((SKB,), jnp.float32),
            pltpu.VMEM((KB, G), jnp.float32),
            pltpu.VMEM((KB, G), jnp.float32),
            pltpu.VMEM_SHARED((NP, G), jnp.float32),
            pltpu.VMEM_SHARED((NP, G), jnp.float32),
            pltpu.SemaphoreType.DMA,
            pltpu.SemaphoreType.DMA,
            pltpu.SemaphoreType.DMA,
            pltpu.SemaphoreType.DMA,
        ],
        compiler_params=pltpu.CompilerParams(use_tc_tiling_on_sc=False),
    )(ego4, col4, row2, val, zz)


BN = 3128


def _dense_body(side_ref, w_ref, b_ref, tot_ref, ego4_ref, totout_ref):
    x = jnp.concatenate([side_ref[0], side_ref[1], side_ref[2], side_ref[3]],
                        axis=1)
    y = x @ w_ref[...] + b_ref[...]
    y = jnp.where(y >= 0, y, 0.2 * y)
    n2 = jnp.sum(y * y, axis=1, keepdims=True)
    nrm = y / jnp.maximum(jnp.sqrt(n2), 1e-12)
    totout_ref[...] = tot_ref[...] + nrm
    for g in range(4):
        ego4_ref[g, :, :] = y[:, g * G:(g + 1) * G]


@jax.jit
def _dense(side4, w, b, total):
    return pl.pallas_call(
        _dense_body,
        grid=(NP // BN,),
        in_specs=[
            pl.BlockSpec((4, BN, G), lambda i: (0, i, 0)),
            pl.BlockSpec((D, D), lambda i: (0, 0)),
            pl.BlockSpec((1, D), lambda i: (0, 0)),
            pl.BlockSpec((BN, D), lambda i: (i, 0)),
        ],
        out_specs=[
            pl.BlockSpec((4, BN, G), lambda i: (0, i, 0)),
            pl.BlockSpec((BN, D), lambda i: (i, 0)),
        ],
        out_shape=[
            jax.ShapeDtypeStruct((4, NP, G), jnp.float32),
            jax.ShapeDtypeStruct((NP, D), jnp.float32),
        ],
    )(side4, w, b, total)


BPW = B // (NC * NS)


def _gather_body(tot_hbm, users_hbm, out_hbm, idxv, rowsv, sem):
    wid = lax.axis_index("s") * NC + lax.axis_index("c")
    base = wid * BPW
    pltpu.sync_copy(users_hbm.at[pl.ds(base, BPW)], idxv)
    pltpu.async_copy(tot_hbm.at[idxv], rowsv, sem).wait()
    pltpu.sync_copy(rowsv, out_hbm.at[pl.ds(base, BPW)])


@jax.jit
def _gather(total, users):
    return pl.kernel(
        _gather_body,
        out_type=jax.ShapeDtypeStruct((B, D), jnp.float32),
        mesh=_mesh,
        scratch_types=[
            pltpu.VMEM((BPW,), jnp.int32),
            pltpu.VMEM((BPW, D), jnp.float32),
            pltpu.SemaphoreType.DMA,
        ],
        compiler_params=pltpu.CompilerParams(use_tc_tiling_on_sc=False),
    )(total, users)


def kernel(users, user_emb, item_emb, adj_row, adj_col, adj_val,
           W_gc_0, b_gc_0, W_gc_1, b_gc_1, W_gc_2, b_gc_2):
    users = users.astype(jnp.int32)
    col = adj_col.astype(jnp.int32)
    row = adj_row.astype(jnp.int32)
    val = adj_val.astype(jnp.float32)

    pad = EPAD - E
    col4 = jnp.concatenate([col, jnp.zeros((pad,), jnp.int32)])
    row2 = jnp.concatenate([row, jnp.full((pad,), N - 1, jnp.int32)])
    valp = jnp.concatenate([val, jnp.zeros((pad,), jnp.float32)])
    zz = jnp.zeros((NR_T, G), jnp.float32)

    ego = jnp.concatenate(
        [user_emb, item_emb, jnp.zeros((NP - N, D), jnp.float32)], axis=0)
    total = ego
    ego4 = jnp.concatenate([ego[:, g * G:(g + 1) * G] for g in range(4)], axis=0)

    for w, bb in ((W_gc_0, b_gc_0), (W_gc_1, b_gc_1), (W_gc_2, b_gc_2)):
        side4 = _spmm(ego4, col4, row2, valp, zz).reshape(4, NP, G)
        ego4n, total = _dense(side4, w, bb, total)
        ego4 = ego4n.reshape(4 * NP, G)

    return _gather(total, users)
